# baseline (speedup 1.0000x reference)
import functools
import jax, jax.numpy as jnp
from jax import lax
from jax.experimental import pallas as pl
from jax.experimental.pallas import tpu as pltpu
from jax.experimental.pallas import tpu_sc as plsc

B = 16384; D = 192

@functools.partial(
    pl.kernel,
    mesh=plsc.VectorSubcoreMesh(core_axis_name="c", subcore_axis_name="s"),
    out_type=jax.ShapeDtypeStruct((8, 128), jnp.float32),
    scratch_types=[pltpu.VMEM((8, 128), jnp.float32), pltpu.SemaphoreType.DMA],
)
def _sc_tiny(giT, o, buf, sem):
    wid = lax.axis_index("s") * 2 + lax.axis_index("c")

    @pl.when(wid == 0)
    def _():
        pltpu.make_async_copy(giT.at[pl.ds(0, 8), pl.ds(0, 128)], buf, sem).start()
        pltpu.make_async_copy(giT.at[pl.ds(0, 8), pl.ds(0, 128)], buf, sem).wait()
        pltpu.sync_copy(buf, o)

def kernel(gu, gi):
    t = _sc_tiny(gi.T)
    return (jnp.zeros((B,), gu.dtype) + t[0, 0], gu, gi)


# speedup vs baseline: 2.3331x; 2.3331x over previous
import jax
import jax.numpy as jnp
from jax.experimental import pallas as pl


def _body(guT_ref, giT_ref, xui_ref, uT_ref, iT_ref):
    u = guT_ref[...]
    v = giT_ref[...]
    uT_ref[...] = u
    iT_ref[...] = v
    xui_ref[...] = jnp.sum(u * v, axis=0)


def kernel(gu, gi):
    B, D = gu.shape
    BS = 2048
    guT = gu.T
    giT = gi.T
    xui, gamma_uT, gamma_iT = pl.pallas_call(
        _body,
        grid=(B // BS,),
        in_specs=[
            pl.BlockSpec((D, BS), lambda b: (0, b)),
            pl.BlockSpec((D, BS), lambda b: (0, b)),
        ],
        out_specs=[
            pl.BlockSpec((BS,), lambda b: (b,)),
            pl.BlockSpec((D, BS), lambda b: (0, b)),
            pl.BlockSpec((D, BS), lambda b: (0, b)),
        ],
        out_shape=[
            jax.ShapeDtypeStruct((B,), gu.dtype),
            jax.ShapeDtypeStruct((D, B), gu.dtype),
            jax.ShapeDtypeStruct((D, B), gi.dtype),
        ],
    )(guT, giT)
    return (xui, gamma_uT.T, gamma_iT.T)


# speedup vs baseline: 2.4951x; 1.0694x over previous
import jax
import jax.numpy as jnp
from jax.experimental import pallas as pl


def _body(guT_ref, giT_ref, xui_ref, uT_ref, iT_ref):
    u = guT_ref[...]
    v = giT_ref[...]
    uT_ref[...] = u
    iT_ref[...] = v
    xui_ref[...] = jnp.sum(u * v, axis=0)


def kernel(gu, gi):
    B, D = gu.shape
    BS = 4096
    guT = gu.T
    giT = gi.T
    xui, gamma_uT, gamma_iT = pl.pallas_call(
        _body,
        grid=(B // BS,),
        in_specs=[
            pl.BlockSpec((D, BS), lambda b: (0, b)),
            pl.BlockSpec((D, BS), lambda b: (0, b)),
        ],
        out_specs=[
            pl.BlockSpec((BS,), lambda b: (b,)),
            pl.BlockSpec((D, BS), lambda b: (0, b)),
            pl.BlockSpec((D, BS), lambda b: (0, b)),
        ],
        out_shape=[
            jax.ShapeDtypeStruct((B,), gu.dtype),
            jax.ShapeDtypeStruct((D, B), gu.dtype),
            jax.ShapeDtypeStruct((D, B), gi.dtype),
        ],
    )(guT, giT)
    return (xui, gamma_uT.T, gamma_iT.T)


# speedup vs baseline: 2.6805x; 1.0743x over previous
import jax
import jax.numpy as jnp
from jax.experimental import pallas as pl


def _body(guT_ref, giT_ref, xui_ref, uT_ref, iT_ref):
    u = guT_ref[...]
    v = giT_ref[...]
    uT_ref[...] = u
    iT_ref[...] = v
    xui_ref[...] = jnp.sum(u * v, axis=0)


def kernel(gu, gi):
    B, D = gu.shape
    BS = 8192
    guT = gu.T
    giT = gi.T
    xui, gamma_uT, gamma_iT = pl.pallas_call(
        _body,
        grid=(B // BS,),
        in_specs=[
            pl.BlockSpec((D, BS), lambda b: (0, b)),
            pl.BlockSpec((D, BS), lambda b: (0, b)),
        ],
        out_specs=[
            pl.BlockSpec((BS,), lambda b: (b,)),
            pl.BlockSpec((D, BS), lambda b: (0, b)),
            pl.BlockSpec((D, BS), lambda b: (0, b)),
        ],
        out_shape=[
            jax.ShapeDtypeStruct((B,), gu.dtype),
            jax.ShapeDtypeStruct((D, B), gu.dtype),
            jax.ShapeDtypeStruct((D, B), gi.dtype),
        ],
    )(guT, giT)
    return (xui, gamma_uT.T, gamma_iT.T)
